# baseline (speedup 1.0000x reference)
import functools

import jax
import jax.numpy as jnp
from jax import lax
from jax.experimental import pallas as pl
from jax.experimental.pallas import tpu as pltpu
from jax.experimental.pallas import tpu_sc as plsc

NC = 2
NS = 16
NW = NC * NS
D = 32
R0, R1 = 4, 8
CHUNK = 128

_MESH = plsc.VectorSubcoreMesh(core_axis_name="c", subcore_axis_name="s")
_NO_TC_TILING = pltpu.CompilerParams(use_tc_tiling_on_sc=False)


def _sc_gather(P3, Q3, ub, ib, uid, iid):
    B = uid.shape[0]
    b_per_w = B // NW
    n_ch = b_per_w // CHUNK

    @functools.partial(
        pl.kernel,
        mesh=_MESH,
        compiler_params=_NO_TC_TILING,
        out_type=(
            jax.ShapeDtypeStruct((B, R0, R1), jnp.float32),
            jax.ShapeDtypeStruct((B, R0, R1), jnp.float32),
            jax.ShapeDtypeStruct((B,), jnp.float32),
            jax.ShapeDtypeStruct((B,), jnp.float32),
        ),
        scratch_types=[
            pltpu.VMEM((b_per_w,), jnp.int32),
            pltpu.VMEM((b_per_w,), jnp.int32),
            pltpu.VMEM((b_per_w, R0, R1), jnp.float32),
            pltpu.VMEM((b_per_w, R0, R1), jnp.float32),
            pltpu.VMEM((b_per_w,), jnp.float32),
            pltpu.VMEM((b_per_w,), jnp.float32),
            pltpu.SemaphoreType.DMA,
            pltpu.SemaphoreType.DMA,
        ],
    )
    def k(P_hbm, Q_hbm, ub_hbm, ib_hbm, uid_hbm, iid_hbm,
          pu_out, qi_out, bu_out, bi_out,
          uid_v, iid_v, pr_v, qr_v, bu_v, bi_v, sem, sem2):
        wid = lax.axis_index("s") * NC + lax.axis_index("c")
        base = wid * b_per_w
        pltpu.sync_copy(uid_hbm.at[pl.ds(base, b_per_w)], uid_v)
        pltpu.sync_copy(iid_hbm.at[pl.ds(base, b_per_w)], iid_v)
        gathers = []
        for c in range(n_ch):
            sl = pl.ds(c * CHUNK, CHUNK)
            gathers.append(pltpu.async_copy(P_hbm.at[uid_v.at[sl]], pr_v.at[sl], sem))
            gathers.append(pltpu.async_copy(Q_hbm.at[iid_v.at[sl]], qr_v.at[sl], sem))
            gathers.append(pltpu.async_copy(ub_hbm.at[uid_v.at[sl]], bu_v.at[sl], sem))
            gathers.append(pltpu.async_copy(ib_hbm.at[iid_v.at[sl]], bi_v.at[sl], sem))
        for g in gathers:
            g.wait()
        sl_out = pl.ds(base, b_per_w)
        outs = [
            pltpu.async_copy(pr_v, pu_out.at[sl_out], sem2),
            pltpu.async_copy(qr_v, qi_out.at[sl_out], sem2),
            pltpu.async_copy(bu_v, bu_out.at[sl_out], sem2),
            pltpu.async_copy(bi_v, bi_out.at[sl_out], sem2),
        ]
        for o in outs:
            o.wait()

    return k(P3, Q3, ub, ib, uid, iid)


def _reduce_body(p_ref, q_ref, bu_ref, bi_ref, o_ref):
    o_ref[...] = (jnp.sum(p_ref[...] * q_ref[...], axis=1)
                  + bu_ref[...] + bi_ref[...])


def _tc_reduce(pu, qi, bu, bi):
    B = pu.shape[0]
    nb = 8
    bb = B // nb
    return pl.pallas_call(
        _reduce_body,
        grid=(nb,),
        in_specs=[
            pl.BlockSpec((bb, D), lambda i: (i, 0)),
            pl.BlockSpec((bb, D), lambda i: (i, 0)),
            pl.BlockSpec((bb,), lambda i: (i,)),
            pl.BlockSpec((bb,), lambda i: (i,)),
        ],
        out_specs=pl.BlockSpec((bb,), lambda i: (i,)),
        out_shape=jax.ShapeDtypeStruct((B,), jnp.float32),
    )(pu, qi, bu, bi)


def kernel(user_id, item_id, P, Q, user_bias, item_bias):
    B = user_id.shape[0]
    P3 = P.reshape(P.shape[0], R0, R1)
    Q3 = Q.reshape(Q.shape[0], R0, R1)
    ub = user_bias.reshape(-1)
    ib = item_bias.reshape(-1)
    pu3, qi3, bu, bi = _sc_gather(P3, Q3, ub, ib, user_id, item_id)
    return _tc_reduce(pu3.reshape(B, D), qi3.reshape(B, D), bu, bi)


# speedup vs baseline: 5.2099x; 5.2099x over previous
import dataclasses
import functools

import jax
import jax.numpy as jnp
from jax import lax
from jax.experimental import pallas as pl
from jax.experimental.pallas import tpu as pltpu
from jax.experimental.pallas import tpu_sc as plsc

NC = 2
NS = 16
NW = NC * NS
D = 32
PACK = 4
ROWW = PACK * D
CHUNK = 128
L = 16

_MESH = plsc.VectorSubcoreMesh(core_axis_name="c", subcore_axis_name="s")
_PARAMS = dataclasses.replace(
    pltpu.CompilerParams(use_tc_tiling_on_sc=False), needs_layout_passes=False)


def _sc_mf(P4, Q4, ub, ib, u4, i4, um, im, uid, iid):
    B = uid.shape[0]
    b_per_w = B // NW
    n_ch = b_per_w // CHUNK
    n_grp = CHUNK // L

    @functools.partial(
        pl.kernel,
        mesh=_MESH,
        compiler_params=_PARAMS,
        out_type=jax.ShapeDtypeStruct((B,), jnp.float32),
        scratch_types=[
            pltpu.VMEM((b_per_w,), jnp.int32),
            pltpu.VMEM((b_per_w,), jnp.int32),
            pltpu.VMEM((b_per_w,), jnp.int32),
            pltpu.VMEM((b_per_w,), jnp.int32),
            pltpu.VMEM((b_per_w,), jnp.int32),
            pltpu.VMEM((b_per_w,), jnp.int32),
            pltpu.VMEM((b_per_w,), jnp.float32),
            pltpu.VMEM((b_per_w,), jnp.float32),
            pltpu.VMEM((b_per_w,), jnp.float32),
            pltpu.VMEM((CHUNK, ROWW), jnp.float32),
            pltpu.VMEM((CHUNK, ROWW), jnp.float32),
            pltpu.VMEM((CHUNK, ROWW), jnp.float32),
            pltpu.VMEM((CHUNK, ROWW), jnp.float32),
            pltpu.SemaphoreType.DMA,
            pltpu.SemaphoreType.DMA,
            pltpu.SemaphoreType.DMA,
        ],
    )
    def k(P_hbm, Q_hbm, ub_hbm, ib_hbm, u4_hbm, i4_hbm, um_hbm, im_hbm,
          uid_hbm, iid_hbm, out_hbm,
          u4_v, i4_v, um_v, im_v, uid_v, iid_v, bu_v, bi_v, out_v,
          dP0, dP1, dQ0, dQ1, semA0, semA1, semB, ):
        wid = lax.axis_index("s") * NC + lax.axis_index("c")
        base = wid * b_per_w
        sl_w = pl.ds(base, b_per_w)
        pltpu.sync_copy(u4_hbm.at[sl_w], u4_v)
        pltpu.sync_copy(i4_hbm.at[sl_w], i4_v)
        pltpu.sync_copy(um_hbm.at[sl_w], um_v)
        pltpu.sync_copy(im_hbm.at[sl_w], im_v)
        pltpu.sync_copy(uid_hbm.at[sl_w], uid_v)
        pltpu.sync_copy(iid_hbm.at[sl_w], iid_v)

        dP = (dP0, dP1)
        dQ = (dQ0, dQ1)
        semA = (semA0, semA1)

        bias_copies = []
        for c in range(n_ch):
            sl = pl.ds(c * CHUNK, CHUNK)
            bias_copies.append(
                pltpu.async_copy(ub_hbm.at[uid_v.at[sl]], bu_v.at[sl], semB))
            bias_copies.append(
                pltpu.async_copy(ib_hbm.at[iid_v.at[sl]], bi_v.at[sl], semB))

        def fire(c):
            sl = pl.ds(c * CHUNK, CHUNK)
            b = c % 2
            return (pltpu.async_copy(P_hbm.at[u4_v.at[sl]], dP[b], semA[b]),
                    pltpu.async_copy(Q_hbm.at[i4_v.at[sl]], dQ[b], semA[b]))

        pend = fire(0)
        for bc in bias_copies:
            bc.wait()

        for c in range(n_ch):
            nxt = fire(c + 1) if c + 1 < n_ch else None
            pend[0].wait()
            pend[1].wait()
            b = c % 2
            dPc, dQc = dP[b], dQ[b]

            @pl.loop(0, n_grp)
            def _(g):
                off = c * CHUNK + g * L
                jrow = lax.iota(jnp.int32, L) + g * L
                cbu = um_v[pl.ds(off, L)] * D
                cbi = im_v[pl.ds(off, L)] * D
                acc0 = bu_v[pl.ds(off, L)] + bi_v[pl.ds(off, L)]

                def body(d8, acc):
                    for t in range(4):
                        d = d8 * 4 + t
                        pc = plsc.load_gather(dPc, [jrow, cbu + d])
                        qc = plsc.load_gather(dQc, [jrow, cbi + d])
                        acc = acc + pc * qc
                    return acc

                out_v[pl.ds(off, L)] = lax.fori_loop(0, 8, body, acc0)

            pend = nxt

        pltpu.sync_copy(out_v, out_hbm.at[sl_w])

    return k(P4, Q4, ub, ib, u4, i4, um, im, uid, iid)


def kernel(user_id, item_id, P, Q, user_bias, item_bias):
    P4 = P.reshape(P.shape[0] // PACK, ROWW)
    Q4 = Q.reshape(Q.shape[0] // PACK, ROWW)
    ub = user_bias.reshape(-1)
    ib = item_bias.reshape(-1)
    u4 = user_id >> 2
    i4 = item_id >> 2
    um = user_id & 3
    im = item_id & 3
    return _sc_mf(P4, Q4, ub, ib, u4, i4, um, im, user_id, item_id)


# speedup vs baseline: 5.2152x; 1.0010x over previous
import dataclasses
import functools

import jax
import jax.numpy as jnp
from jax import lax
from jax.experimental import pallas as pl
from jax.experimental.pallas import tpu as pltpu
from jax.experimental.pallas import tpu_sc as plsc

NC = 2
NS = 16
NW = NC * NS
D = 32
PACK = 4
ROWW = PACK * D
CHUNK = 128
L = 16

_MESH = plsc.VectorSubcoreMesh(core_axis_name="c", subcore_axis_name="s")
_PARAMS = dataclasses.replace(
    pltpu.CompilerParams(), needs_layout_passes=False)


def _sc_mf(P4, Q4, ub, ib, u4, i4, um, im, uid, iid):
    B = uid.shape[0]
    b_per_w = B // NW
    n_ch = b_per_w // CHUNK
    n_grp = CHUNK // L

    @functools.partial(
        pl.kernel,
        mesh=_MESH,
        compiler_params=_PARAMS,
        out_type=jax.ShapeDtypeStruct((B,), jnp.float32),
        scratch_types=[
            pltpu.VMEM((b_per_w,), jnp.int32),
            pltpu.VMEM((b_per_w,), jnp.int32),
            pltpu.VMEM((b_per_w,), jnp.int32),
            pltpu.VMEM((b_per_w,), jnp.int32),
            pltpu.VMEM((b_per_w,), jnp.int32),
            pltpu.VMEM((b_per_w,), jnp.int32),
            pltpu.VMEM((b_per_w,), jnp.float32),
            pltpu.VMEM((b_per_w,), jnp.float32),
            pltpu.VMEM((b_per_w,), jnp.float32),
            pltpu.VMEM((CHUNK, ROWW), jnp.float32),
            pltpu.VMEM((CHUNK, ROWW), jnp.float32),
            pltpu.VMEM((CHUNK, ROWW), jnp.float32),
            pltpu.VMEM((CHUNK, ROWW), jnp.float32),
            pltpu.SemaphoreType.DMA,
            pltpu.SemaphoreType.DMA,
            pltpu.SemaphoreType.DMA,
        ],
    )
    def k(P_hbm, Q_hbm, ub_hbm, ib_hbm, u4_hbm, i4_hbm, um_hbm, im_hbm,
          uid_hbm, iid_hbm, out_hbm,
          u4_v, i4_v, um_v, im_v, uid_v, iid_v, bu_v, bi_v, out_v,
          dP0, dP1, dQ0, dQ1, semA0, semA1, semB, ):
        wid = lax.axis_index("s") * NC + lax.axis_index("c")
        base = wid * b_per_w
        sl_w = pl.ds(base, b_per_w)
        pltpu.sync_copy(u4_hbm.at[sl_w], u4_v)
        pltpu.sync_copy(i4_hbm.at[sl_w], i4_v)
        pltpu.sync_copy(um_hbm.at[sl_w], um_v)
        pltpu.sync_copy(im_hbm.at[sl_w], im_v)
        pltpu.sync_copy(uid_hbm.at[sl_w], uid_v)
        pltpu.sync_copy(iid_hbm.at[sl_w], iid_v)

        dP = (dP0, dP1)
        dQ = (dQ0, dQ1)
        semA = (semA0, semA1)

        bias_copies = []
        for c in range(n_ch):
            sl = pl.ds(c * CHUNK, CHUNK)
            bias_copies.append(
                pltpu.async_copy(ub_hbm.at[uid_v.at[sl]], bu_v.at[sl], semB))
            bias_copies.append(
                pltpu.async_copy(ib_hbm.at[iid_v.at[sl]], bi_v.at[sl], semB))

        def fire(c):
            sl = pl.ds(c * CHUNK, CHUNK)
            b = c % 2
            return (pltpu.async_copy(P_hbm.at[u4_v.at[sl]], dP[b], semA[b]),
                    pltpu.async_copy(Q_hbm.at[i4_v.at[sl]], dQ[b], semA[b]))

        pend = fire(0)
        for bc in bias_copies:
            bc.wait()

        for c in range(n_ch):
            nxt = fire(c + 1) if c + 1 < n_ch else None
            pend[0].wait()
            pend[1].wait()
            b = c % 2
            dPc, dQc = dP[b], dQ[b]

            @pl.loop(0, n_grp)
            def _(g):
                off = c * CHUNK + g * L
                jrow = lax.iota(jnp.int32, L) + g * L
                cbu = um_v[pl.ds(off, L)] * D
                cbi = im_v[pl.ds(off, L)] * D
                acc0 = bu_v[pl.ds(off, L)] + bi_v[pl.ds(off, L)]

                def body(d8, acc):
                    for t in range(4):
                        d = d8 * 4 + t
                        pc = plsc.load_gather(dPc, [jrow, cbu + d])
                        qc = plsc.load_gather(dQc, [jrow, cbi + d])
                        acc = acc + pc * qc
                    return acc

                out_v[pl.ds(off, L)] = lax.fori_loop(0, 8, body, acc0)

            pend = nxt

        pltpu.sync_copy(out_v, out_hbm.at[sl_w])

    return k(P4, Q4, ub, ib, u4, i4, um, im, uid, iid)


def kernel(user_id, item_id, P, Q, user_bias, item_bias):
    P4 = P.reshape(P.shape[0] // PACK, ROWW)
    Q4 = Q.reshape(Q.shape[0] // PACK, ROWW)
    ub = user_bias.reshape(-1)
    ib = item_bias.reshape(-1)
    u4 = user_id >> 2
    i4 = item_id >> 2
    um = user_id & 3
    im = item_id & 3
    return _sc_mf(P4, Q4, ub, ib, u4, i4, um, im, user_id, item_id)
